# baseline (speedup 1.0000x reference)
import numpy as np
import jax
import jax.numpy as jnp
from jax.experimental import pallas as pl
from jax.experimental.pallas import tpu as pltpu

_EPS = 1e-10
_S = 16
_K = 16
_PIX_BLOCK = 512

_ERFINV_SMALL = [2.81022636e-08, 3.43273939e-07, -3.5233877e-06,
                 -4.39150654e-06, 0.00021858087, -0.00125372503,
                 -0.00417768164, 0.246640727, 1.50140941]
_ERFINV_LARGE = [-0.000200214257, 0.000100950558, 0.00134934322,
                 -0.00367342844, 0.00573950773, -0.0076224613,
                 0.00943887047, 1.00167406, 2.83297682]

_SQRT2 = np.float32(np.sqrt(2.0))
_ULO = np.nextafter(np.float32(-1.0), np.float32(0.0), dtype=np.float32)
_USCALE = np.float32(np.float32(1.0) - _ULO)


def _threefry_bits(idx):
    ks0 = jnp.uint32(0)
    ks1 = jnp.uint32(1)
    ks2 = jnp.uint32(0x1BD11BDA) ^ ks0 ^ ks1

    def rounds(x0, x1, rots):
        for r in rots:
            x0 = x0 + x1
            x1 = (x1 << jnp.uint32(r)) | (x1 >> jnp.uint32(32 - r))
            x1 = x0 ^ x1
        return x0, x1

    x0 = jnp.zeros_like(idx) + ks0
    x1 = idx + ks1
    x0, x1 = rounds(x0, x1, (13, 15, 26, 6))
    x0 = x0 + ks1
    x1 = x1 + ks2 + jnp.uint32(1)
    x0, x1 = rounds(x0, x1, (17, 29, 16, 24))
    x0 = x0 + ks2
    x1 = x1 + ks0 + jnp.uint32(2)
    x0, x1 = rounds(x0, x1, (13, 15, 26, 6))
    x0 = x0 + ks0
    x1 = x1 + ks1 + jnp.uint32(3)
    x0, x1 = rounds(x0, x1, (17, 29, 16, 24))
    x0 = x0 + ks1
    x1 = x1 + ks2 + jnp.uint32(4)
    x0, x1 = rounds(x0, x1, (13, 15, 26, 6))
    x0 = x0 + ks2
    x1 = x1 + ks0 + jnp.uint32(5)
    return x0 ^ x1


def _bits_to_normal(bits):
    fb = (bits >> jnp.uint32(9)) | jnp.uint32(0x3F800000)
    f = jax.lax.bitcast_convert_type(fb, jnp.float32) - jnp.float32(1.0)
    u = jnp.maximum(_ULO, f * _USCALE + _ULO)
    w = -jnp.log1p(-u * u)
    ws = w - jnp.float32(2.5)
    wl = jnp.sqrt(w) - jnp.float32(3.0)
    ps = jnp.float32(_ERFINV_SMALL[0])
    for c in _ERFINV_SMALL[1:]:
        ps = jnp.float32(c) + ps * ws
    pl_ = jnp.float32(_ERFINV_LARGE[0])
    for c in _ERFINV_LARGE[1:]:
        pl_ = jnp.float32(c) + pl_ * wl
    p = jnp.where(w < jnp.float32(5.0), ps, pl_)
    return _SQRT2 * (p * u)


def _gauss_agg_kernel(sc_ref, zb_ref, pm_ref, mk_ref, out_ref):
    P = _PIX_BLOCK
    gamma = sc_ref[0]
    alpha = sc_ref[1]
    zfar = sc_ref[2]
    znear = sc_ref[3]

    zb = zb_ref[...]
    pm = pm_ref[...]
    mk = mk_ref[...]

    z_inv = (zfar - zb) / (zfar - znear) * mk
    z_inv_max = jnp.maximum(jnp.max(z_inv, axis=0, keepdims=True),
                            jnp.float32(_EPS))
    zmap = (gamma / alpha) * jnp.log(pm) + z_inv - z_inv_max
    zbg = jnp.float32(_EPS) - z_inv_max

    pix0 = pl.program_id(0) * P
    lane16 = jax.lax.broadcasted_iota(jnp.uint32, (_K, P), 1)
    chan16 = jax.lax.broadcasted_iota(jnp.uint32, (_K, P), 0)
    base16 = (jnp.uint32(pix0) + lane16) * jnp.uint32(17) + chan16
    lane1 = jax.lax.broadcasted_iota(jnp.uint32, (1, P), 1)
    basebg = (jnp.uint32(pix0) + lane1) * jnp.uint32(17) + jnp.uint32(16)

    iota_c = jax.lax.broadcasted_iota(jnp.int32, (_K, P), 0)
    acc16 = jnp.zeros((_K, P), jnp.float32)
    accbg = jnp.zeros((1, P), jnp.float32)
    stride_s = jnp.uint32(4 * 256 * 256 * 17)
    for s in range(_S):
        off = jnp.uint32(s) * stride_s
        zp = zmap + gamma * _bits_to_normal(_threefry_bits(base16 + off))
        zpbg = zbg + gamma * _bits_to_normal(_threefry_bits(basebg + off))
        m16 = jnp.max(zp, axis=0, keepdims=True)
        bgwin = zpbg > m16
        cidx = jnp.min(jnp.where(zp == m16, iota_c, jnp.int32(_K)),
                       axis=0, keepdims=True)
        win16 = (iota_c == cidx) & jnp.logical_not(bgwin)
        acc16 = acc16 + jnp.where(win16, jnp.float32(1.0), jnp.float32(0.0))
        accbg = accbg + jnp.where(bgwin, jnp.float32(1.0), jnp.float32(0.0))

    inv_s = jnp.float32(1.0 / _S)
    out_ref[0:_K, :] = acc16 * inv_s
    out_ref[_K:_K + 1, :] = accbg * inv_s


def kernel(zbuf, prob_map, mask, gamma, alpha, zfar, znear):
    B, H, W, K = zbuf.shape
    N = B * H * W
    zb_t = zbuf.reshape(N, K).T
    pm_t = prob_map.reshape(N, K).T
    mk_t = mask.reshape(N, K).T
    scal = jnp.stack([gamma[0], alpha[0], zfar[0], znear[0]]).astype(jnp.float32)

    grid = (N // _PIX_BLOCK,)
    out_t = pl.pallas_call(
        _gauss_agg_kernel,
        grid=grid,
        in_specs=[
            pl.BlockSpec(memory_space=pltpu.SMEM),
            pl.BlockSpec((K, _PIX_BLOCK), lambda i: (0, i)),
            pl.BlockSpec((K, _PIX_BLOCK), lambda i: (0, i)),
            pl.BlockSpec((K, _PIX_BLOCK), lambda i: (0, i)),
        ],
        out_specs=pl.BlockSpec((K + 1, _PIX_BLOCK), lambda i: (0, i)),
        out_shape=jax.ShapeDtypeStruct((K + 1, N), jnp.float32),
        compiler_params=pltpu.CompilerParams(
            dimension_semantics=("parallel",)),
    )(scal, zb_t, pm_t, mk_t)
    return out_t.T.reshape(B, H, W, K + 1)


# speedup vs baseline: 1.3833x; 1.3833x over previous
import numpy as np
import jax
import jax.numpy as jnp
from jax.experimental import pallas as pl
from jax.experimental.pallas import tpu as pltpu

_EPS = 1e-10
_K = 16
_PIX_BLOCK = 512
_SUB = 128
_STRIDE_S = 4 * 256 * 256 * 17

_ERFINV_SMALL = [2.81022636e-08, 3.43273939e-07, -3.5233877e-06,
                 -4.39150654e-06, 0.00021858087, -0.00125372503,
                 -0.00417768164, 0.246640727, 1.50140941]
_ERFINV_LARGE = [-0.000200214257, 0.000100950558, 0.00134934322,
                 -0.00367342844, 0.00573950773, -0.0076224613,
                 0.00943887047, 1.00167406, 2.83297682]

_SQRT2 = np.float32(np.sqrt(2.0))
_ULO = np.nextafter(np.float32(-1.0), np.float32(0.0), dtype=np.float32)
_USCALE = np.float32(np.float32(1.0) - _ULO)


def _threefry_normal(x1):
    ks0 = jnp.uint32(0)
    ks1 = jnp.uint32(1)
    ks2 = jnp.uint32(0x1BD11BDA) ^ ks0 ^ ks1

    def rounds(x0, x1, rots):
        for r in rots:
            x0 = x0 + x1
            x1 = (x1 << jnp.uint32(r)) | (x1 >> jnp.uint32(32 - r))
            x1 = x0 ^ x1
        return x0, x1

    x0 = x1
    x1 = ((x1 << jnp.uint32(13)) | (x1 >> jnp.uint32(19))) ^ x0
    x0, x1 = rounds(x0, x1, (15, 26, 6))
    x0 = x0 + ks1
    x1 = x1 + ks2 + jnp.uint32(1)
    x0, x1 = rounds(x0, x1, (17, 29, 16, 24))
    x0 = x0 + ks2
    x1 = x1 + ks0 + jnp.uint32(2)
    x0, x1 = rounds(x0, x1, (13, 15, 26, 6))
    x0 = x0 + ks0
    x1 = x1 + ks1 + jnp.uint32(3)
    x0, x1 = rounds(x0, x1, (17, 29, 16, 24))
    x0 = x0 + ks1
    x1 = x1 + ks2 + jnp.uint32(4)
    x0, x1 = rounds(x0, x1, (13, 15, 26, 6))
    x0 = x0 + ks2
    x1 = x1 + ks0 + jnp.uint32(5)
    bits = x0 ^ x1

    fb = (bits >> jnp.uint32(9)) | jnp.uint32(0x3F800000)
    f = jax.lax.bitcast_convert_type(fb, jnp.float32) - jnp.float32(1.0)
    u = jnp.maximum(_ULO, f * _USCALE + _ULO)
    w = -jnp.log1p(-u * u)
    ws = w - jnp.float32(2.5)
    wl = jnp.sqrt(w) - jnp.float32(3.0)
    ps = jnp.float32(_ERFINV_SMALL[0])
    for c in _ERFINV_SMALL[1:]:
        ps = jnp.float32(c) + ps * ws
    pl_ = jnp.float32(_ERFINV_LARGE[0])
    for c in _ERFINV_LARGE[1:]:
        pl_ = jnp.float32(c) + pl_ * wl
    p = jnp.where(w < jnp.float32(5.0), ps, pl_)
    return _SQRT2 * (p * u)


def _gauss_agg_kernel(sc_ref, zb_ref, pm_ref, mk_ref, out_ref):
    gamma = sc_ref[0]
    alpha = sc_ref[1]
    zfar = sc_ref[2]
    znear = sc_ref[3]

    pix0 = pl.program_id(0) * _PIX_BLOCK
    lane17 = (jax.lax.broadcasted_iota(jnp.uint32, (8, _SUB), 1)
              * jnp.uint32(17))
    sstride = (jax.lax.broadcasted_iota(jnp.uint32, (8, _SUB), 0)
               * jnp.uint32(_STRIDE_S))
    base = lane17 + sstride

    for sub in range(_PIX_BLOCK // _SUB):
        sl = slice(sub * _SUB, (sub + 1) * _SUB)
        zb = zb_ref[:, sl]
        pm = pm_ref[:, sl]
        mk = mk_ref[:, sl]

        z_inv = (zfar - zb) / (zfar - znear) * mk
        z_inv_max = jnp.maximum(jnp.max(z_inv, axis=0, keepdims=True),
                                jnp.float32(_EPS))
        zmap = (gamma / alpha) * jnp.log(pm) + z_inv - z_inv_max
        zbg = jnp.float32(_EPS) - z_inv_max

        zt = [jnp.broadcast_to(zmap[c:c + 1, :], (8, _SUB))
              for c in range(_K)]
        zt.append(jnp.broadcast_to(zbg, (8, _SUB)))

        cidx_g = []
        for g in range(2):
            scal = pix0 * 17 + sub * _SUB * 17 + g * 8 * _STRIDE_S
            ctr = base + jnp.uint32(scal)
            zp = []
            for c in range(_K + 1):
                noise = _threefry_normal(ctr + jnp.uint32(c + 1))
                zp.append(zt[c] + gamma * noise)
            m = zp[0]
            for c in range(1, _K + 1):
                m = jnp.maximum(m, zp[c])
            cidx = jnp.full((8, _SUB), _K, jnp.int32)
            for c in range(_K - 1, -1, -1):
                cidx = jnp.where(zp[c] == m, jnp.int32(c), cidx)
            cidx_g.append(cidx)

        rows = []
        inv_s = jnp.float32(1.0 / 16.0)
        for c in range(_K + 1):
            tot = (jnp.where(cidx_g[0] == c, jnp.float32(1.0), jnp.float32(0.0))
                   + jnp.where(cidx_g[1] == c, jnp.float32(1.0),
                               jnp.float32(0.0)))
            rows.append(jnp.sum(tot, axis=0, keepdims=True) * inv_s)
        out_ref[:, sl] = jnp.concatenate(rows, axis=0)


def kernel(zbuf, prob_map, mask, gamma, alpha, zfar, znear):
    B, H, W, K = zbuf.shape
    N = B * H * W
    zb_t = zbuf.reshape(N, K).T
    pm_t = prob_map.reshape(N, K).T
    mk_t = mask.reshape(N, K).T
    scal = jnp.stack([gamma[0], alpha[0], zfar[0], znear[0]]).astype(jnp.float32)

    grid = (N // _PIX_BLOCK,)
    out_t = pl.pallas_call(
        _gauss_agg_kernel,
        grid=grid,
        in_specs=[
            pl.BlockSpec(memory_space=pltpu.SMEM),
            pl.BlockSpec((K, _PIX_BLOCK), lambda i: (0, i)),
            pl.BlockSpec((K, _PIX_BLOCK), lambda i: (0, i)),
            pl.BlockSpec((K, _PIX_BLOCK), lambda i: (0, i)),
        ],
        out_specs=pl.BlockSpec((K + 1, _PIX_BLOCK), lambda i: (0, i)),
        out_shape=jax.ShapeDtypeStruct((K + 1, N), jnp.float32),
        compiler_params=pltpu.CompilerParams(
            dimension_semantics=("parallel",)),
    )(scal, zb_t, pm_t, mk_t)
    return out_t.T.reshape(B, H, W, K + 1)


# speedup vs baseline: 1.3963x; 1.0094x over previous
import numpy as np
import jax
import jax.numpy as jnp
from jax.experimental import pallas as pl
from jax.experimental.pallas import tpu as pltpu

_EPS = 1e-10
_K = 16
_PIX_BLOCK = 1024
_SUB = 128
_STRIDE_S = 4 * 256 * 256 * 17

_ERFINV_SMALL = [2.81022636e-08, 3.43273939e-07, -3.5233877e-06,
                 -4.39150654e-06, 0.00021858087, -0.00125372503,
                 -0.00417768164, 0.246640727, 1.50140941]
_ERFINV_LARGE = [-0.000200214257, 0.000100950558, 0.00134934322,
                 -0.00367342844, 0.00573950773, -0.0076224613,
                 0.00943887047, 1.00167406, 2.83297682]

_SQRT2 = np.float32(np.sqrt(2.0))
_ULO = np.nextafter(np.float32(-1.0), np.float32(0.0), dtype=np.float32)
_USCALE = np.float32(np.float32(1.0) - _ULO)


def _threefry_normal(x1):
    ks0 = jnp.uint32(0)
    ks1 = jnp.uint32(1)
    ks2 = jnp.uint32(0x1BD11BDA) ^ ks0 ^ ks1

    def rounds(x0, x1, rots):
        for r in rots:
            x0 = x0 + x1
            x1 = (x1 << jnp.uint32(r)) | (x1 >> jnp.uint32(32 - r))
            x1 = x0 ^ x1
        return x0, x1

    x0 = x1
    x1 = ((x1 << jnp.uint32(13)) | (x1 >> jnp.uint32(19))) ^ x0
    x0, x1 = rounds(x0, x1, (15, 26, 6))
    x0 = x0 + ks1
    x1 = x1 + ks2 + jnp.uint32(1)
    x0, x1 = rounds(x0, x1, (17, 29, 16, 24))
    x0 = x0 + ks2
    x1 = x1 + ks0 + jnp.uint32(2)
    x0, x1 = rounds(x0, x1, (13, 15, 26, 6))
    x0 = x0 + ks0
    x1 = x1 + ks1 + jnp.uint32(3)
    x0, x1 = rounds(x0, x1, (17, 29, 16, 24))
    x0 = x0 + ks1
    x1 = x1 + ks2 + jnp.uint32(4)
    x0, x1 = rounds(x0, x1, (13, 15, 26, 6))
    x0 = x0 + ks2
    x1 = x1 + ks0 + jnp.uint32(5)
    bits = x0 ^ x1

    fb = (bits >> jnp.uint32(9)) | jnp.uint32(0x3F800000)
    f = jax.lax.bitcast_convert_type(fb, jnp.float32) - jnp.float32(1.0)
    u = jnp.maximum(_ULO, f * _USCALE + _ULO)
    w = -jnp.log1p(-u * u)
    ws = w - jnp.float32(2.5)
    wl = jnp.sqrt(w) - jnp.float32(3.0)
    ps = jnp.float32(_ERFINV_SMALL[0])
    for c in _ERFINV_SMALL[1:]:
        ps = jnp.float32(c) + ps * ws
    pl_ = jnp.float32(_ERFINV_LARGE[0])
    for c in _ERFINV_LARGE[1:]:
        pl_ = jnp.float32(c) + pl_ * wl
    p = jnp.where(w < jnp.float32(5.0), ps, pl_)
    return _SQRT2 * (p * u)


def _gauss_agg_kernel(sc_ref, zb_ref, pm_ref, mk_ref, out_ref):
    gamma = sc_ref[0]
    alpha = sc_ref[1]
    zfar = sc_ref[2]
    znear = sc_ref[3]

    pix0 = pl.program_id(0) * _PIX_BLOCK
    lane17 = (jax.lax.broadcasted_iota(jnp.uint32, (8, _SUB), 1)
              * jnp.uint32(17))
    sstride = (jax.lax.broadcasted_iota(jnp.uint32, (8, _SUB), 0)
               * jnp.uint32(_STRIDE_S))
    base = lane17 + sstride

    for sub in range(_PIX_BLOCK // _SUB):
        sl = slice(sub * _SUB, (sub + 1) * _SUB)
        zb = zb_ref[:, sl]
        pm = pm_ref[:, sl]
        mk = mk_ref[:, sl]

        z_inv = (zfar - zb) / (zfar - znear) * mk
        z_inv_max = jnp.maximum(jnp.max(z_inv, axis=0, keepdims=True),
                                jnp.float32(_EPS))
        zmap = (gamma / alpha) * jnp.log(pm) + z_inv - z_inv_max
        zbg = jnp.float32(_EPS) - z_inv_max

        zt = [jnp.broadcast_to(zmap[c:c + 1, :], (8, _SUB))
              for c in range(_K)]
        zt.append(jnp.broadcast_to(zbg, (8, _SUB)))

        cidx_g = []
        for g in range(2):
            scal = pix0 * 17 + sub * _SUB * 17 + g * 8 * _STRIDE_S
            ctr = base + jnp.uint32(scal)
            m = zt[0] + gamma * _threefry_normal(ctr + jnp.uint32(1))
            cidx = jnp.zeros((8, _SUB), jnp.int32)
            for c in range(1, _K + 1):
                noise = _threefry_normal(ctr + jnp.uint32(c + 1))
                zpc = zt[c] + gamma * noise
                gt = zpc > m
                m = jnp.maximum(m, zpc)
                cidx = jnp.where(gt, jnp.int32(c), cidx)
            cidx_g.append(cidx)

        rows = []
        inv_s = jnp.float32(1.0 / 16.0)
        for c in range(_K + 1):
            tot = (jnp.where(cidx_g[0] == c, jnp.float32(1.0), jnp.float32(0.0))
                   + jnp.where(cidx_g[1] == c, jnp.float32(1.0),
                               jnp.float32(0.0)))
            rows.append(jnp.sum(tot, axis=0, keepdims=True) * inv_s)
        out_ref[:, sl] = jnp.concatenate(rows, axis=0)


def kernel(zbuf, prob_map, mask, gamma, alpha, zfar, znear):
    B, H, W, K = zbuf.shape
    N = B * H * W
    zb_t = zbuf.reshape(N, K).T
    pm_t = prob_map.reshape(N, K).T
    mk_t = mask.reshape(N, K).T
    scal = jnp.stack([gamma[0], alpha[0], zfar[0], znear[0]]).astype(jnp.float32)

    grid = (N // _PIX_BLOCK,)
    out_t = pl.pallas_call(
        _gauss_agg_kernel,
        grid=grid,
        in_specs=[
            pl.BlockSpec(memory_space=pltpu.SMEM),
            pl.BlockSpec((K, _PIX_BLOCK), lambda i: (0, i)),
            pl.BlockSpec((K, _PIX_BLOCK), lambda i: (0, i)),
            pl.BlockSpec((K, _PIX_BLOCK), lambda i: (0, i)),
        ],
        out_specs=pl.BlockSpec((K + 1, _PIX_BLOCK), lambda i: (0, i)),
        out_shape=jax.ShapeDtypeStruct((K + 1, N), jnp.float32),
        compiler_params=pltpu.CompilerParams(
            dimension_semantics=("parallel",)),
    )(scal, zb_t, pm_t, mk_t)
    return out_t.T.reshape(B, H, W, K + 1)
